# baseline (speedup 1.0000x reference)
import functools
import math

import jax
import jax.numpy as jnp
from jax import lax
from jax.experimental import pallas as pl
from jax.experimental.pallas import tpu as pltpu
from jax.experimental.pallas import tpu_sc as plsc

_D = 128
_LANES = 16
_NW = 32
_NSEG = 100
_NCLS = 2


def _pick_block(sizes, cap, step):
    for b in range(cap - cap % step, 0, -step):
        if all(sz % b == 0 for sz in sizes):
            return b
    raise ValueError(f"no block size <= {cap} divides {sizes}")


def _starts(sizes):
    out, s = [], 0
    for sz in sizes:
        out.append(s)
        s += sz
    return out


def _nbr_reduce(x, idx_ts, op):
    n = x.shape[0]
    degs = [t.shape[0] for t in idx_ts]
    sizes = [t.shape[1] for t in idx_ts]
    starts = _starts(sizes)
    blk = _pick_block(sizes, _D, 8)
    idx_ts = [jnp.pad(t.reshape(t.shape[0], t.shape[1] // blk, blk),
                      ((0, 0), (0, 0), (0, _D - blk)))
              for t in idx_ts]
    maxd = max(degs)
    nbuf_max = maxd if op == "max" else maxd - 1

    mesh = plsc.VectorSubcoreMesh(core_axis_name="c", subcore_axis_name="s",
                                  num_cores=2, num_subcores=16)
    scratch = ([pltpu.VMEM((_D,), jnp.int32) for _ in range(2 * maxd)]
               + [pltpu.VMEM((blk, _D), jnp.float32)
                  for _ in range(2 * max(nbuf_max, 1))]
               + [pltpu.VMEM((blk, _D), jnp.float32),
                  pltpu.VMEM((blk, _D), jnp.float32),
                  pltpu.SemaphoreType.DMA, pltpu.SemaphoreType.DMA,
                  pltpu.SemaphoreType.DMA, pltpu.SemaphoreType.DMA])

    @functools.partial(
        pl.kernel,
        out_type=jax.ShapeDtypeStruct((n, _D), jnp.float32),
        mesh=mesh,
        scratch_types=scratch,
    )
    def knl(x_hbm, *rest):
        nidx = len(idx_ts)
        idx_refs = rest[:nidx]
        out_hbm = rest[nidx]
        rest = rest[nidx + 1:]
        nb = max(nbuf_max, 1)
        idxs = [rest[:maxd], rest[maxd:2 * maxd]]
        bufs = [rest[2 * maxd:2 * maxd + nb],
                rest[2 * maxd + nb:2 * maxd + 2 * nb]]
        accs = rest[2 * maxd + 2 * nb:2 * maxd + 2 * nb + 2]
        sem_g = rest[2 * maxd + 2 * nb + 2:2 * maxd + 2 * nb + 4]
        sem_w = rest[2 * maxd + 2 * nb + 4:2 * maxd + 2 * nb + 6]
        wid = lax.axis_index("s") * 2 + lax.axis_index("c")

        for di in range(nidx):
            d, g0, nblk = degs[di], starts[di], sizes[di] // blk
            idx_t = idx_refs[di]
            nmine = (nblk - wid + (_NW - 1)) // _NW
            nbuf = d if op == "max" else d - 1

            def fetch_idx(j, p, d=d, idx_t=idx_t):
                bidx = wid + _NW * j
                cps = [pltpu.async_copy(idx_t.at[kk, bidx],
                                        idxs[p][kk], sem_g[p])
                       for kk in range(d)]
                for cp in cps:
                    cp.wait()

            def gather_dsts(p, d=d):
                if op == "max":
                    return [accs[p]] + [bufs[p][kk] for kk in range(d)]
                return [accs[p]] + [bufs[p][kk] for kk in range(d - 1)]

            def issue_gathers(j, p, d=d, g0=g0):
                off = (wid + _NW * j) * blk
                ivs = [idxs[p][kk].at[pl.ds(0, blk)] for kk in range(d)]
                if op == "max":
                    pltpu.async_copy(x_hbm.at[pl.ds(g0 + off, blk)],
                                     accs[p], sem_g[p])
                    for kk in range(d):
                        pltpu.async_copy(x_hbm.at[ivs[kk]], bufs[p][kk],
                                         sem_g[p])
                else:
                    pltpu.async_copy(x_hbm.at[ivs[0]], accs[p], sem_g[p])
                    for kk in range(1, d):
                        pltpu.async_copy(x_hbm.at[ivs[kk]], bufs[p][kk - 1],
                                         sem_g[p])

            def drain_gathers(p, d=d):
                for dst in gather_dsts(p):
                    pltpu.make_async_copy(x_hbm.at[pl.ds(0, blk)], dst,
                                          sem_g[p]).wait()

            def drain_write(p):
                pltpu.make_async_copy(accs[p], out_hbm.at[pl.ds(0, blk)],
                                      sem_w[p]).wait()

            @pl.when(nmine > 0)
            def prologue():
                fetch_idx(0, 0)
                issue_gathers(0, 0)

            def body(j, carry, d=d, g0=g0, nbuf=nbuf):
                p = lax.rem(j, 2)

                def for_parity(fn):
                    @pl.when(p == 0)
                    def _():
                        fn(0)
                    @pl.when(p == 1)
                    def _():
                        fn(1)

                @pl.when(j + 1 < nmine)
                def prefetch_idx():
                    for_parity(lambda q: fetch_idx(j + 1, 1 - q))

                for_parity(drain_gathers)

                @pl.when(j >= 1)
                def drain_prev_write():
                    for_parity(lambda q: drain_write(1 - q))

                @pl.when(j + 1 < nmine)
                def launch_next():
                    for_parity(lambda q: issue_gathers(j + 1, 1 - q))

                if nbuf:
                    def rbody(r, rc):
                        for c in range(_D // _LANES):
                            sl = pl.ds(c * _LANES, _LANES)
                            def red(q, r=r, sl=sl):
                                v = accs[q][r, sl]
                                for kb in range(nbuf):
                                    if op == "max":
                                        v = jnp.maximum(v, bufs[q][kb][r, sl])
                                    else:
                                        v = v + bufs[q][kb][r, sl]
                                accs[q][r, sl] = v
                            for_parity(red)
                        return rc
                    lax.fori_loop(0, blk, rbody, 0)

                off = (wid + _NW * j) * blk
                for_parity(lambda q: pltpu.async_copy(
                    accs[q], out_hbm.at[pl.ds(g0 + off, blk)], sem_w[q])
                    and None)
                return carry

            lax.fori_loop(0, nmine, body, 0)

            @pl.when(nmine > 0)
            def epilogue():
                pl0 = lax.rem(nmine - 1, 2)
                @pl.when(pl0 == 0)
                def _():
                    drain_write(0)
                @pl.when(pl0 == 1)
                def _():
                    drain_write(1)

    return knl(x, *idx_ts)


def _seg_partials(xd, memb):
    n = xd.shape[0]
    tile = _pick_block([n], 160, 8)
    chunk = -(-n // (_NW * tile)) * tile
    mrow = -(-tile // _D) * _D
    memb = jnp.pad(memb.reshape(n // tile, tile), ((0, 0), (0, mrow - tile)))

    mesh = plsc.VectorSubcoreMesh(core_axis_name="c", subcore_axis_name="s", num_cores=2, num_subcores=16)
    out_t = (jax.ShapeDtypeStruct((_NW, _NSEG, _D), jnp.float32),
             jax.ShapeDtypeStruct((_NW, _NSEG, _D), jnp.float32))
    scratch = [
        pltpu.VMEM((tile, _D), jnp.float32),
        pltpu.VMEM((mrow,), jnp.int32),
        pltpu.VMEM((_NSEG, _D), jnp.float32),
        pltpu.VMEM((_NSEG, _D), jnp.float32),
        pltpu.SemaphoreType.DMA,
    ]

    @functools.partial(pl.kernel, out_type=out_t, mesh=mesh,
                       scratch_types=scratch)
    def knl(x_hbm, m_hbm, ps_hbm, pm_hbm, xv, mv, accs, accm, sem):
        wid = lax.axis_index("s") * 2 + lax.axis_index("c")
        base = wid * chunk
        cnt = jnp.maximum(jnp.minimum(chunk, n - base), 0)
        ntile = cnt // tile

        def init_body(r, c0):
            for c in range(_D // _LANES):
                sl = pl.ds(c * _LANES, _LANES)
                accs[r, sl] = jnp.zeros((_LANES,), jnp.float32)
                accm[r, sl] = jnp.full((_LANES,), -jnp.inf, jnp.float32)
            return c0
        lax.fori_loop(0, _NSEG, init_body, 0)

        def tbody(t, c0):
            r0 = base + t * tile
            cp1 = pltpu.async_copy(x_hbm.at[pl.ds(r0, tile)], xv, sem)
            cp2 = pltpu.async_copy(m_hbm.at[wid * (chunk // tile) + t],
                                   mv, sem)
            cp1.wait()
            cp2.wait()

            def rbody(r, rc):
                s = mv[pl.ds(r, _LANES)][0]
                for c in range(_D // _LANES):
                    sl = pl.ds(c * _LANES, _LANES)
                    v = xv[r, sl]
                    plsc.addupdate(accs.at[s, sl], v)
                    accm[s, sl] = jnp.maximum(accm[s, sl], v)
                return rc
            lax.fori_loop(0, tile, rbody, 0)
            return c0
        lax.fori_loop(0, ntile, tbody, 0)

        pltpu.sync_copy(accs, ps_hbm.at[wid])
        pltpu.sync_copy(accm, pm_hbm.at[wid])

    return knl(xd, memb)


def _conv_tc(x, rel, wcat, bmat, sizes):
    n = x.shape[0]
    tb = _pick_block(sizes, 2048, 8)
    nblk = n // tb
    cum, acc = [], 0
    for sz in sizes[:-1]:
        acc += sz // tb
        cum.append(acc)

    def dmap(i):
        t = jnp.int32(0)
        for cb in cum:
            t = t + (i >= cb).astype(jnp.int32)
        return t

    def body(xr, rr, wr, br, orf):
        cat = jnp.concatenate([xr[...], rr[...]], axis=1)
        y = jnp.dot(cat, wr[0], preferred_element_type=jnp.float32)
        orf[...] = jnp.maximum(y + br[0], 0.0)

    return pl.pallas_call(
        body,
        grid=(nblk,),
        in_specs=[
            pl.BlockSpec((tb, _D), lambda i: (i, 0)),
            pl.BlockSpec((tb, _D), lambda i: (i, 0)),
            pl.BlockSpec((1, 2 * _D, _D), lambda i: (dmap(i), 0, 0)),
            pl.BlockSpec((1, 1, _D), lambda i: (dmap(i), 0, 0)),
        ],
        out_specs=pl.BlockSpec((tb, _D), lambda i: (i, 0)),
        out_shape=jax.ShapeDtypeStruct((n, _D), jnp.float32),
    )(x, rel, wcat, bmat.reshape(-1, 1, _D))


def _dense_tc(x, w, b):
    n = x.shape[0]
    tb = _pick_block([n], 2048, 8)

    def body(xr, wr, br, orf):
        y = jnp.dot(xr[...], wr[...], preferred_element_type=jnp.float32)
        orf[...] = jnp.maximum(y + br[...], 0.0)

    return pl.pallas_call(
        body,
        grid=(n // tb,),
        in_specs=[
            pl.BlockSpec((tb, _D), lambda i: (i, 0)),
            pl.BlockSpec((_D, _D), lambda i: (0, 0)),
            pl.BlockSpec((1, _D), lambda i: (0, 0)),
        ],
        out_specs=pl.BlockSpec((tb, _D), lambda i: (i, 0)),
        out_shape=jax.ShapeDtypeStruct((n, _D), jnp.float32),
    )(x, w, b)


def _head_tc(psum, pmax, w0, b0, w1, b1, w2, b2, nsm):
    nt = w2.shape[1]

    def body(ps, pm, w0r, b0r, w1r, b1r, w2r, b2r, nr,
             o_out, o_lg, o_fp):
        sums = jnp.sum(ps[...], axis=0)
        mx = jnp.max(pm[...], axis=0)
        fp = jnp.tanh(jnp.concatenate([sums, mx], axis=1))
        h = jnp.dot(fp, w0r[...], preferred_element_type=jnp.float32)
        h = jnp.maximum(h + b0r[...], 0.0)
        h = jnp.dot(h, w1r[...], preferred_element_type=jnp.float32)
        h = jnp.maximum(h + b1r[...], 0.0)
        lg = jnp.dot(h, w2r[...], preferred_element_type=jnp.float32) + b2r[...]
        rmask = lax.broadcasted_iota(jnp.int32, (_NSEG, nt), 0) < nr[0, 0]
        lg = jnp.where(rmask, lg, 0.0)
        ii = lax.broadcasted_iota(jnp.int32, (nt, nt), 0)
        jj = lax.broadcasted_iota(jnp.int32, (nt, nt), 1)
        sw = (ii == (jj ^ 1)).astype(jnp.float32)
        lsw = jnp.dot(lg, sw, preferred_element_type=jnp.float32)
        m = jnp.maximum(lg, lsw)
        e = jnp.exp(lg - m)
        esw = jnp.dot(e, sw, preferred_element_type=jnp.float32)
        o_out[...] = e / (e + esw)
        o_lg[...] = lg
        o_fp[...] = fp

    return pl.pallas_call(
        body,
        out_shape=(
            jax.ShapeDtypeStruct((_NSEG, nt), jnp.float32),
            jax.ShapeDtypeStruct((_NSEG, nt), jnp.float32),
            jax.ShapeDtypeStruct((_NSEG, 2 * _D), jnp.float32),
        ),
    )(psum, pmax, w0, b0, w1, b1, w2, b2, nsm)


def kernel(atom_features, degree_slice, membership, n_samples,
           deg_adj_1, deg_adj_2, deg_adj_3, deg_adj_4, deg_adj_5,
           gc0_W, gc0_b, gc1_W, gc1_b, dense_W, dense_b,
           fd0_W, fd0_b, fd1_W, fd1_b, out_W, out_b):
    adjs = [deg_adj_1, deg_adj_2, deg_adj_3, deg_adj_4, deg_adj_5]
    sizes = [a.shape[0] for a in adjs]
    idx_ts = [a.T for a in adjs]

    def wcat(w):
        return jnp.stack([jnp.concatenate([w[2 * i], w[2 * i + 1]], axis=0)
                          for i in range(5)])

    w0, b0 = wcat(gc0_W), gc0_b[1:6]
    w1, b1 = wcat(gc1_W), gc1_b[1:6]

    rel0 = _nbr_reduce(atom_features, idx_ts, "sum")
    h1 = _conv_tc(atom_features, rel0, w0, b0, sizes)
    p1 = _nbr_reduce(h1, idx_ts, "max")
    rel1 = _nbr_reduce(p1, idx_ts, "sum")
    h2 = _conv_tc(p1, rel1, w1, b1, sizes)
    p2 = _nbr_reduce(h2, idx_ts, "max")
    dn = _dense_tc(p2, dense_W, dense_b.reshape(1, -1))
    ps, pm = _seg_partials(dn, membership)
    nsm = jnp.asarray(n_samples, jnp.int32).reshape(1, 1)
    out24, lg24, fp = _head_tc(
        ps, pm, fd0_W, fd0_b.reshape(1, -1), fd1_W, fd1_b.reshape(1, -1),
        out_W, out_b.reshape(1, -1), nsm)
    nt = out_W.shape[1]
    return (out24.reshape(_NSEG, nt // _NCLS, _NCLS),
            lg24.reshape(_NSEG, nt // _NCLS, _NCLS),
            fp)


# speedup vs baseline: 1.4376x; 1.4376x over previous
import functools
import math

import jax
import jax.numpy as jnp
from jax import lax
from jax.experimental import pallas as pl
from jax.experimental.pallas import tpu as pltpu
from jax.experimental.pallas import tpu_sc as plsc

_D = 128
_LANES = 16
_NW = 32
_NSEG = 100
_NCLS = 2


def _pick_block(sizes, cap, step):
    for b in range(cap - cap % step, 0, -step):
        if all(sz % b == 0 for sz in sizes):
            return b
    raise ValueError(f"no block size <= {cap} divides {sizes}")


def _starts(sizes):
    out, s = [], 0
    for sz in sizes:
        out.append(s)
        s += sz
    return out


def _nbr_reduce(x, idx_ts, op):
    n = x.shape[0]
    degs = [t.shape[0] for t in idx_ts]
    sizes = [t.shape[1] for t in idx_ts]
    starts = _starts(sizes)
    blk = _pick_block(sizes, _D, 8)
    idx_ts = [jnp.pad(t.reshape(t.shape[0], t.shape[1] // blk, blk),
                      ((0, 0), (0, 0), (0, _D - blk)))
              for t in idx_ts]
    maxd = max(degs)
    nbuf_max = maxd if op == "max" else maxd - 1

    mesh = plsc.VectorSubcoreMesh(core_axis_name="c", subcore_axis_name="s",
                                  num_cores=2, num_subcores=16)
    scratch = ([pltpu.VMEM((_D,), jnp.int32) for _ in range(2 * maxd)]
               + [pltpu.VMEM((blk, _D), jnp.float32)
                  for _ in range(2 * max(nbuf_max, 1))]
               + [pltpu.VMEM((blk, _D), jnp.float32),
                  pltpu.VMEM((blk, _D), jnp.float32),
                  pltpu.SemaphoreType.DMA, pltpu.SemaphoreType.DMA,
                  pltpu.SemaphoreType.DMA, pltpu.SemaphoreType.DMA])

    @functools.partial(
        pl.kernel,
        out_type=jax.ShapeDtypeStruct((n, _D), jnp.float32),
        mesh=mesh,
        scratch_types=scratch,
    )
    def knl(x_hbm, *rest):
        nidx = len(idx_ts)
        idx_refs = rest[:nidx]
        out_hbm = rest[nidx]
        rest = rest[nidx + 1:]
        nb = max(nbuf_max, 1)
        idxs = [rest[:maxd], rest[maxd:2 * maxd]]
        bufs = [rest[2 * maxd:2 * maxd + nb],
                rest[2 * maxd + nb:2 * maxd + 2 * nb]]
        accs = rest[2 * maxd + 2 * nb:2 * maxd + 2 * nb + 2]
        sem_g = rest[2 * maxd + 2 * nb + 2:2 * maxd + 2 * nb + 4]
        sem_w = rest[2 * maxd + 2 * nb + 4:2 * maxd + 2 * nb + 6]
        wid = lax.axis_index("s") * 2 + lax.axis_index("c")

        for di in range(nidx):
            d, g0, nblk = degs[di], starts[di], sizes[di] // blk
            idx_t = idx_refs[di]
            nmine = (nblk - wid + (_NW - 1)) // _NW
            nbuf = d if op == "max" else d - 1

            def fetch_idx(j, p, d=d, idx_t=idx_t):
                bidx = wid + _NW * j
                cps = [pltpu.async_copy(idx_t.at[kk, bidx],
                                        idxs[p][kk], sem_g[p])
                       for kk in range(d)]
                for cp in cps:
                    cp.wait()

            def gather_dsts(p, d=d):
                if op == "max":
                    return [accs[p]] + [bufs[p][kk] for kk in range(d)]
                return [accs[p]] + [bufs[p][kk] for kk in range(d - 1)]

            def issue_gathers(j, p, d=d, g0=g0):
                off = (wid + _NW * j) * blk
                ivs = [idxs[p][kk].at[pl.ds(0, blk)] for kk in range(d)]
                if op == "max":
                    pltpu.async_copy(x_hbm.at[pl.ds(g0 + off, blk)],
                                     accs[p], sem_g[p])
                    for kk in range(d):
                        pltpu.async_copy(x_hbm.at[ivs[kk]], bufs[p][kk],
                                         sem_g[p])
                else:
                    pltpu.async_copy(x_hbm.at[ivs[0]], accs[p], sem_g[p])
                    for kk in range(1, d):
                        pltpu.async_copy(x_hbm.at[ivs[kk]], bufs[p][kk - 1],
                                         sem_g[p])

            def drain_gathers(p, d=d):
                for dst in gather_dsts(p):
                    pltpu.make_async_copy(x_hbm.at[pl.ds(0, blk)], dst,
                                          sem_g[p]).wait()

            def drain_write(p):
                pltpu.make_async_copy(accs[p], out_hbm.at[pl.ds(0, blk)],
                                      sem_w[p]).wait()

            def reduce_block(q, nbuf=nbuf):
                if not nbuf:
                    return
                def rbody(r, rc):
                    for c in range(_D // _LANES):
                        sl = pl.ds(c * _LANES, _LANES)
                        v = accs[q][r, sl]
                        for kb in range(nbuf):
                            if op == "max":
                                v = jnp.maximum(v, bufs[q][kb][r, sl])
                            else:
                                v = v + bufs[q][kb][r, sl]
                        accs[q][r, sl] = v
                    return rc
                lax.fori_loop(0, blk, rbody, 0)

            def write_block(j, q, g0=g0):
                off = (wid + _NW * j) * blk
                pltpu.async_copy(accs[q], out_hbm.at[pl.ds(g0 + off, blk)],
                                 sem_w[q])

            @pl.when(nmine > 0)
            def prologue():
                fetch_idx(0, 0)
                issue_gathers(0, 0)

            def pbody(t, carry):
                j0 = 2 * t
                j1 = j0 + 1
                @pl.when(j1 < nmine)
                def _():
                    fetch_idx(j1, 1)
                drain_gathers(0)
                @pl.when(j0 >= 1)
                def _():
                    drain_write(1)
                @pl.when(j1 < nmine)
                def _():
                    issue_gathers(j1, 1)
                reduce_block(0)
                write_block(j0, 0)
                @pl.when(j1 + 1 < nmine)
                def _():
                    fetch_idx(j1 + 1, 0)
                drain_gathers(1)
                drain_write(0)
                @pl.when(j1 + 1 < nmine)
                def _():
                    issue_gathers(j1 + 1, 0)
                reduce_block(1)
                write_block(j1, 1)
                return carry

            lax.fori_loop(0, nmine // 2, pbody, 0)

            @pl.when(lax.rem(nmine, 2) == 1)
            def tail():
                drain_gathers(0)
                @pl.when(nmine > 1)
                def _():
                    drain_write(1)
                reduce_block(0)
                write_block(nmine - 1, 0)

            @pl.when(nmine > 0)
            def epilogue():
                pl0 = lax.rem(nmine - 1, 2)
                @pl.when(pl0 == 0)
                def _():
                    drain_write(0)
                @pl.when(pl0 == 1)
                def _():
                    drain_write(1)

    return knl(x, *idx_ts)


def _seg_partials(xd, memb):
    n = xd.shape[0]
    tile = _pick_block([n], 160, 8)
    chunk = -(-n // (_NW * tile)) * tile
    mrow = -(-tile // _D) * _D
    memb = jnp.pad(memb.reshape(n // tile, tile), ((0, 0), (0, mrow - tile)))

    mesh = plsc.VectorSubcoreMesh(core_axis_name="c", subcore_axis_name="s", num_cores=2, num_subcores=16)
    out_t = (jax.ShapeDtypeStruct((_NW, _NSEG, _D), jnp.float32),
             jax.ShapeDtypeStruct((_NW, _NSEG, _D), jnp.float32))
    scratch = [
        pltpu.VMEM((tile, _D), jnp.float32),
        pltpu.VMEM((mrow,), jnp.int32),
        pltpu.VMEM((_NSEG, _D), jnp.float32),
        pltpu.VMEM((_NSEG, _D), jnp.float32),
        pltpu.SemaphoreType.DMA,
    ]

    @functools.partial(pl.kernel, out_type=out_t, mesh=mesh,
                       scratch_types=scratch)
    def knl(x_hbm, m_hbm, ps_hbm, pm_hbm, xv, mv, accs, accm, sem):
        wid = lax.axis_index("s") * 2 + lax.axis_index("c")
        base = wid * chunk
        cnt = jnp.maximum(jnp.minimum(chunk, n - base), 0)
        ntile = cnt // tile

        def init_body(r, c0):
            for c in range(_D // _LANES):
                sl = pl.ds(c * _LANES, _LANES)
                accs[r, sl] = jnp.zeros((_LANES,), jnp.float32)
                accm[r, sl] = jnp.full((_LANES,), -jnp.inf, jnp.float32)
            return c0
        lax.fori_loop(0, _NSEG, init_body, 0)

        def tbody(t, c0):
            r0 = base + t * tile
            cp1 = pltpu.async_copy(x_hbm.at[pl.ds(r0, tile)], xv, sem)
            cp2 = pltpu.async_copy(m_hbm.at[wid * (chunk // tile) + t],
                                   mv, sem)
            cp1.wait()
            cp2.wait()

            def rbody(r, rc):
                s = mv[pl.ds(r, _LANES)][0]
                for c in range(_D // _LANES):
                    sl = pl.ds(c * _LANES, _LANES)
                    v = xv[r, sl]
                    plsc.addupdate(accs.at[s, sl], v)
                    accm[s, sl] = jnp.maximum(accm[s, sl], v)
                return rc
            lax.fori_loop(0, tile, rbody, 0)
            return c0
        lax.fori_loop(0, ntile, tbody, 0)

        pltpu.sync_copy(accs, ps_hbm.at[wid])
        pltpu.sync_copy(accm, pm_hbm.at[wid])

    return knl(xd, memb)


def _conv_tc(x, rel, wcat, bmat, sizes):
    n = x.shape[0]
    tb = _pick_block(sizes, 2048, 8)
    nblk = n // tb
    cum, acc = [], 0
    for sz in sizes[:-1]:
        acc += sz // tb
        cum.append(acc)

    def dmap(i):
        t = jnp.int32(0)
        for cb in cum:
            t = t + (i >= cb).astype(jnp.int32)
        return t

    def body(xr, rr, wr, br, orf):
        cat = jnp.concatenate([xr[...], rr[...]], axis=1)
        y = jnp.dot(cat, wr[0], preferred_element_type=jnp.float32)
        orf[...] = jnp.maximum(y + br[0], 0.0)

    return pl.pallas_call(
        body,
        grid=(nblk,),
        in_specs=[
            pl.BlockSpec((tb, _D), lambda i: (i, 0)),
            pl.BlockSpec((tb, _D), lambda i: (i, 0)),
            pl.BlockSpec((1, 2 * _D, _D), lambda i: (dmap(i), 0, 0)),
            pl.BlockSpec((1, 1, _D), lambda i: (dmap(i), 0, 0)),
        ],
        out_specs=pl.BlockSpec((tb, _D), lambda i: (i, 0)),
        out_shape=jax.ShapeDtypeStruct((n, _D), jnp.float32),
    )(x, rel, wcat, bmat.reshape(-1, 1, _D))


def _dense_tc(x, w, b):
    n = x.shape[0]
    tb = _pick_block([n], 2048, 8)

    def body(xr, wr, br, orf):
        y = jnp.dot(xr[...], wr[...], preferred_element_type=jnp.float32)
        orf[...] = jnp.maximum(y + br[...], 0.0)

    return pl.pallas_call(
        body,
        grid=(n // tb,),
        in_specs=[
            pl.BlockSpec((tb, _D), lambda i: (i, 0)),
            pl.BlockSpec((_D, _D), lambda i: (0, 0)),
            pl.BlockSpec((1, _D), lambda i: (0, 0)),
        ],
        out_specs=pl.BlockSpec((tb, _D), lambda i: (i, 0)),
        out_shape=jax.ShapeDtypeStruct((n, _D), jnp.float32),
    )(x, w, b)


def _head_tc(psum, pmax, w0, b0, w1, b1, w2, b2, nsm):
    nt = w2.shape[1]

    def body(ps, pm, w0r, b0r, w1r, b1r, w2r, b2r, nr,
             o_out, o_lg, o_fp):
        sums = jnp.sum(ps[...], axis=0)
        mx = jnp.max(pm[...], axis=0)
        fp = jnp.tanh(jnp.concatenate([sums, mx], axis=1))
        h = jnp.dot(fp, w0r[...], preferred_element_type=jnp.float32)
        h = jnp.maximum(h + b0r[...], 0.0)
        h = jnp.dot(h, w1r[...], preferred_element_type=jnp.float32)
        h = jnp.maximum(h + b1r[...], 0.0)
        lg = jnp.dot(h, w2r[...], preferred_element_type=jnp.float32) + b2r[...]
        rmask = lax.broadcasted_iota(jnp.int32, (_NSEG, nt), 0) < nr[0, 0]
        lg = jnp.where(rmask, lg, 0.0)
        ii = lax.broadcasted_iota(jnp.int32, (nt, nt), 0)
        jj = lax.broadcasted_iota(jnp.int32, (nt, nt), 1)
        sw = (ii == (jj ^ 1)).astype(jnp.float32)
        lsw = jnp.dot(lg, sw, preferred_element_type=jnp.float32)
        m = jnp.maximum(lg, lsw)
        e = jnp.exp(lg - m)
        esw = jnp.dot(e, sw, preferred_element_type=jnp.float32)
        o_out[...] = e / (e + esw)
        o_lg[...] = lg
        o_fp[...] = fp

    return pl.pallas_call(
        body,
        out_shape=(
            jax.ShapeDtypeStruct((_NSEG, nt), jnp.float32),
            jax.ShapeDtypeStruct((_NSEG, nt), jnp.float32),
            jax.ShapeDtypeStruct((_NSEG, 2 * _D), jnp.float32),
        ),
    )(psum, pmax, w0, b0, w1, b1, w2, b2, nsm)


def kernel(atom_features, degree_slice, membership, n_samples,
           deg_adj_1, deg_adj_2, deg_adj_3, deg_adj_4, deg_adj_5,
           gc0_W, gc0_b, gc1_W, gc1_b, dense_W, dense_b,
           fd0_W, fd0_b, fd1_W, fd1_b, out_W, out_b):
    adjs = [deg_adj_1, deg_adj_2, deg_adj_3, deg_adj_4, deg_adj_5]
    sizes = [a.shape[0] for a in adjs]
    idx_ts = [a.T for a in adjs]

    def wcat(w):
        return jnp.stack([jnp.concatenate([w[2 * i], w[2 * i + 1]], axis=0)
                          for i in range(5)])

    w0, b0 = wcat(gc0_W), gc0_b[1:6]
    w1, b1 = wcat(gc1_W), gc1_b[1:6]

    rel0 = _nbr_reduce(atom_features, idx_ts, "sum")
    h1 = _conv_tc(atom_features, rel0, w0, b0, sizes)
    p1 = _nbr_reduce(h1, idx_ts, "max")
    rel1 = _nbr_reduce(p1, idx_ts, "sum")
    h2 = _conv_tc(p1, rel1, w1, b1, sizes)
    p2 = _nbr_reduce(h2, idx_ts, "max")
    dn = _dense_tc(p2, dense_W, dense_b.reshape(1, -1))
    ps, pm = _seg_partials(dn, membership)
    nsm = jnp.asarray(n_samples, jnp.int32).reshape(1, 1)
    out24, lg24, fp = _head_tc(
        ps, pm, fd0_W, fd0_b.reshape(1, -1), fd1_W, fd1_b.reshape(1, -1),
        out_W, out_b.reshape(1, -1), nsm)
    nt = out_W.shape[1]
    return (out24.reshape(_NSEG, nt // _NCLS, _NCLS),
            lg24.reshape(_NSEG, nt // _NCLS, _NCLS),
            fp)


# speedup vs baseline: 1.6119x; 1.1212x over previous
import functools
import math

import jax
import jax.numpy as jnp
from jax import lax
from jax.experimental import pallas as pl
from jax.experimental.pallas import tpu as pltpu
from jax.experimental.pallas import tpu_sc as plsc

_D = 128
_LANES = 16
_NW = 32
_NSEG = 100
_NCLS = 2


def _pick_block(sizes, cap, step):
    for b in range(cap - cap % step, 0, -step):
        if all(sz % b == 0 for sz in sizes):
            return b
    raise ValueError(f"no block size <= {cap} divides {sizes}")


def _starts(sizes):
    out, s = [], 0
    for sz in sizes:
        out.append(s)
        s += sz
    return out


def _nbr_reduce(x, idx_ts, op):
    n = x.shape[0]
    degs = [t.shape[0] for t in idx_ts]
    sizes = [t.shape[1] for t in idx_ts]
    starts = _starts(sizes)
    blk = _pick_block(sizes, _D, 8)
    idx_ts = [jnp.pad(t.reshape(t.shape[0], t.shape[1] // blk, blk),
                      ((0, 0), (0, 0), (0, _D - blk)))
              for t in idx_ts]
    maxd = max(degs)
    nbuf_max = maxd if op == "max" else maxd - 1

    mesh = plsc.VectorSubcoreMesh(core_axis_name="c", subcore_axis_name="s",
                                  num_cores=2, num_subcores=16)
    scratch = ([pltpu.VMEM((_D,), jnp.int32) for _ in range(2 * maxd)]
               + [pltpu.VMEM((blk, _D), jnp.float32)
                  for _ in range(2 * max(nbuf_max, 1))]
               + [pltpu.VMEM((blk, _D), jnp.float32),
                  pltpu.VMEM((blk, _D), jnp.float32),
                  pltpu.SemaphoreType.DMA, pltpu.SemaphoreType.DMA,
                  pltpu.SemaphoreType.DMA, pltpu.SemaphoreType.DMA])

    @functools.partial(
        pl.kernel,
        out_type=jax.ShapeDtypeStruct((n, _D), jnp.float32),
        mesh=mesh,
        scratch_types=scratch,
    )
    def knl(x_hbm, *rest):
        nidx = len(idx_ts)
        idx_refs = rest[:nidx]
        out_hbm = rest[nidx]
        rest = rest[nidx + 1:]
        nb = max(nbuf_max, 1)
        idxs = [rest[:maxd], rest[maxd:2 * maxd]]
        bufs = [rest[2 * maxd:2 * maxd + nb],
                rest[2 * maxd + nb:2 * maxd + 2 * nb]]
        accs = rest[2 * maxd + 2 * nb:2 * maxd + 2 * nb + 2]
        sem_g = rest[2 * maxd + 2 * nb + 2:2 * maxd + 2 * nb + 4]
        sem_w = rest[2 * maxd + 2 * nb + 4:2 * maxd + 2 * nb + 6]
        wid = lax.axis_index("s") * 2 + lax.axis_index("c")

        for di in range(nidx):
            d, g0, nblk = degs[di], starts[di], sizes[di] // blk
            idx_t = idx_refs[di]
            nmine = (nblk - wid + (_NW - 1)) // _NW
            nbuf = d if op == "max" else d - 1

            def fetch_idx(j, p, d=d, idx_t=idx_t):
                bidx = wid + _NW * j
                cps = [pltpu.async_copy(idx_t.at[kk, bidx],
                                        idxs[p][kk], sem_g[p])
                       for kk in range(d)]
                for cp in cps:
                    cp.wait()

            def gather_dsts(p, d=d):
                if op == "max":
                    return [accs[p]] + [bufs[p][kk] for kk in range(d)]
                return [accs[p]] + [bufs[p][kk] for kk in range(d - 1)]

            def issue_gathers(j, p, d=d, g0=g0):
                off = (wid + _NW * j) * blk
                ivs = [idxs[p][kk].at[pl.ds(0, blk)] for kk in range(d)]
                if op == "max":
                    pltpu.async_copy(x_hbm.at[pl.ds(g0 + off, blk)],
                                     accs[p], sem_g[p])
                    for kk in range(d):
                        pltpu.async_copy(x_hbm.at[ivs[kk]], bufs[p][kk],
                                         sem_g[p])
                else:
                    pltpu.async_copy(x_hbm.at[ivs[0]], accs[p], sem_g[p])
                    for kk in range(1, d):
                        pltpu.async_copy(x_hbm.at[ivs[kk]], bufs[p][kk - 1],
                                         sem_g[p])

            def drain_gathers(p, d=d):
                for dst in gather_dsts(p):
                    pltpu.make_async_copy(x_hbm.at[pl.ds(0, blk)], dst,
                                          sem_g[p]).wait()

            def drain_write(p):
                pltpu.make_async_copy(accs[p], out_hbm.at[pl.ds(0, blk)],
                                      sem_w[p]).wait()

            def reduce_block(q, nbuf=nbuf):
                if not nbuf:
                    return
                def rbody(r, rc):
                    for c in range(_D // _LANES):
                        sl = pl.ds(c * _LANES, _LANES)
                        v = accs[q][r, sl]
                        for kb in range(nbuf):
                            if op == "max":
                                v = jnp.maximum(v, bufs[q][kb][r, sl])
                            else:
                                v = v + bufs[q][kb][r, sl]
                        accs[q][r, sl] = v
                    return rc
                lax.fori_loop(0, blk, rbody, 0)

            def write_block(j, q, g0=g0):
                off = (wid + _NW * j) * blk
                pltpu.async_copy(accs[q], out_hbm.at[pl.ds(g0 + off, blk)],
                                 sem_w[q])

            @pl.when(nmine > 0)
            def prologue():
                fetch_idx(0, 0)
                issue_gathers(0, 0)

            def pbody(t, carry):
                j0 = 2 * t
                j1 = j0 + 1
                @pl.when(j1 < nmine)
                def _():
                    fetch_idx(j1, 1)
                drain_gathers(0)
                @pl.when(j0 >= 1)
                def _():
                    drain_write(1)
                @pl.when(j1 < nmine)
                def _():
                    issue_gathers(j1, 1)
                reduce_block(0)
                write_block(j0, 0)
                @pl.when(j1 + 1 < nmine)
                def _():
                    fetch_idx(j1 + 1, 0)
                drain_gathers(1)
                drain_write(0)
                @pl.when(j1 + 1 < nmine)
                def _():
                    issue_gathers(j1 + 1, 0)
                reduce_block(1)
                write_block(j1, 1)
                return carry

            lax.fori_loop(0, nmine // 2, pbody, 0)

            @pl.when(lax.rem(nmine, 2) == 1)
            def tail():
                drain_gathers(0)
                @pl.when(nmine > 1)
                def _():
                    drain_write(1)
                reduce_block(0)
                write_block(nmine - 1, 0)

            @pl.when(nmine > 0)
            def epilogue():
                pl0 = lax.rem(nmine - 1, 2)
                @pl.when(pl0 == 0)
                def _():
                    drain_write(0)
                @pl.when(pl0 == 1)
                def _():
                    drain_write(1)

    return knl(x, *idx_ts)


def _seg_partials(xd, memb):
    n = xd.shape[0]
    tile = _pick_block([n], 160, 8)
    chunk = -(-n // (_NW * tile)) * tile
    mrow = -(-tile // _D) * _D
    memb = jnp.pad(memb.reshape(n // tile, tile), ((0, 0), (0, mrow - tile)))

    mesh = plsc.VectorSubcoreMesh(core_axis_name="c", subcore_axis_name="s", num_cores=2, num_subcores=16)
    out_t = (jax.ShapeDtypeStruct((_NW, _NSEG, _D), jnp.float32),
             jax.ShapeDtypeStruct((_NW, _NSEG, _D), jnp.float32))
    nlan = _D // _LANES
    scratch = [
        pltpu.VMEM((2, tile, _D), jnp.float32),
        pltpu.VMEM((mrow,), jnp.int32),
        pltpu.VMEM((mrow,), jnp.int32),
        pltpu.VMEM((_NSEG, _D), jnp.float32),
        pltpu.VMEM((_NSEG, _D), jnp.float32),
        pltpu.VMEM((_D,), jnp.float32),
        pltpu.VMEM((_D,), jnp.float32),
        pltpu.VMEM((_LANES,), jnp.int32),
        pltpu.SemaphoreType.DMA, pltpu.SemaphoreType.DMA,
    ]

    @functools.partial(pl.kernel, out_type=out_t, mesh=mesh,
                       scratch_types=scratch)
    def knl(x_hbm, m_hbm, ps_hbm, pm_hbm, xv, mv0, mv1, accs, accm,
            sreg, mreg, curv, s0, s1):
        sems = [s0, s1]
        mvs = [mv0, mv1]
        wid = lax.axis_index("s") * 2 + lax.axis_index("c")
        base = wid * chunk
        cnt = jnp.maximum(jnp.minimum(chunk, n - base), 0)
        ntile = cnt // tile

        def init_body(r, c0):
            for c in range(nlan):
                sl = pl.ds(c * _LANES, _LANES)
                accs[r, sl] = jnp.zeros((_LANES,), jnp.float32)
                accm[r, sl] = jnp.full((_LANES,), -jnp.inf, jnp.float32)
            return c0
        lax.fori_loop(0, _NSEG, init_body, 0)

        def issue(t, p):
            pltpu.async_copy(x_hbm.at[pl.ds(base + t * tile, tile)],
                             xv.at[p], sems[p])
            pltpu.async_copy(m_hbm.at[wid * (chunk // tile) + t],
                             mvs[p], sems[p])

        def drain(p):
            pltpu.make_async_copy(x_hbm.at[pl.ds(0, tile)], xv.at[p],
                                  sems[p]).wait()
            pltpu.make_async_copy(m_hbm.at[0], mvs[p], sems[p]).wait()

        def flush(cur, regs):
            ss, mm = regs
            for c in range(nlan):
                sl = pl.ds(c * _LANES, _LANES)
                plsc.addupdate(accs.at[cur, sl], ss[c])
                accm[cur, sl] = jnp.maximum(accm[cur, sl], mm[c])

        @pl.when(ntile > 0)
        def prologue():
            issue(0, 0)

        zero = jnp.zeros((_LANES,), jnp.float32)
        ninf = jnp.full((_LANES,), -jnp.inf, jnp.float32)
        curv[pl.ds(0, _LANES)] = jnp.full((_LANES,), -1, jnp.int32)
        for c in range(nlan):
            sreg[pl.ds(c * _LANES, _LANES)] = zero
            mreg[pl.ds(c * _LANES, _LANES)] = ninf

        def load_regs():
            cur = curv[pl.ds(0, _LANES)][0]
            ss = [sreg[pl.ds(c * _LANES, _LANES)] for c in range(nlan)]
            mm = [mreg[pl.ds(c * _LANES, _LANES)] for c in range(nlan)]
            return cur, ss, mm

        def store_regs(cur, ss, mm):
            curv[pl.ds(0, _LANES)] = jnp.full((_LANES,), 1, jnp.int32) * cur
            for c in range(nlan):
                sreg[pl.ds(c * _LANES, _LANES)] = ss[c]
                mreg[pl.ds(c * _LANES, _LANES)] = mm[c]

        def inner(q):
            cur0, ss0, mm0 = load_regs()

            def rbody(r, rc):
                cur = rc[0]
                ss = rc[1:1 + nlan]
                mm = rc[1 + nlan:]
                s = mvs[q][pl.ds(r, _LANES)][0]
                chg = s != cur

                @pl.when(jnp.logical_and(chg, cur >= 0))
                def _():
                    flush(cur, (ss, mm))

                nss, nmm = [], []
                for c in range(nlan):
                    v = xv[q, r, pl.ds(c * _LANES, _LANES)]
                    nss.append(jnp.where(chg, v, ss[c] + v))
                    nmm.append(jnp.where(chg, v, jnp.maximum(mm[c], v)))
                return (s,) + tuple(nss) + tuple(nmm)

            fin = lax.fori_loop(0, tile, rbody,
                                (cur0,) + tuple(ss0) + tuple(mm0))
            store_regs(fin[0], fin[1:1 + nlan], fin[1 + nlan:])

        def tbody(t, carry):
            p = lax.rem(t, 2)

            @pl.when(t + 1 < ntile)
            def _():
                @pl.when(p == 0)
                def _():
                    issue(t + 1, 1)
                @pl.when(p == 1)
                def _():
                    issue(t + 1, 0)

            @pl.when(p == 0)
            def _():
                drain(0)
                inner(0)

            @pl.when(p == 1)
            def _():
                drain(1)
                inner(1)

            return carry

        lax.fori_loop(0, ntile, tbody, 0)

        fcur, fss, fmm = load_regs()

        @pl.when(fcur >= 0)
        def _():
            flush(fcur, (fss, fmm))

        pltpu.sync_copy(accs, ps_hbm.at[wid])
        pltpu.sync_copy(accm, pm_hbm.at[wid])

    return knl(xd, memb)


def _conv_tc(x, rel, wcat, bmat, sizes):
    n = x.shape[0]
    tb = _pick_block(sizes, 2048, 8)
    nblk = n // tb
    cum, acc = [], 0
    for sz in sizes[:-1]:
        acc += sz // tb
        cum.append(acc)

    def dmap(i):
        t = jnp.int32(0)
        for cb in cum:
            t = t + (i >= cb).astype(jnp.int32)
        return t

    def body(xr, rr, wr, br, orf):
        cat = jnp.concatenate([xr[...], rr[...]], axis=1)
        y = jnp.dot(cat, wr[0], preferred_element_type=jnp.float32)
        orf[...] = jnp.maximum(y + br[0], 0.0)

    return pl.pallas_call(
        body,
        grid=(nblk,),
        in_specs=[
            pl.BlockSpec((tb, _D), lambda i: (i, 0)),
            pl.BlockSpec((tb, _D), lambda i: (i, 0)),
            pl.BlockSpec((1, 2 * _D, _D), lambda i: (dmap(i), 0, 0)),
            pl.BlockSpec((1, 1, _D), lambda i: (dmap(i), 0, 0)),
        ],
        out_specs=pl.BlockSpec((tb, _D), lambda i: (i, 0)),
        out_shape=jax.ShapeDtypeStruct((n, _D), jnp.float32),
    )(x, rel, wcat, bmat.reshape(-1, 1, _D))


def _dense_tc(x, w, b):
    n = x.shape[0]
    tb = _pick_block([n], 2048, 8)

    def body(xr, wr, br, orf):
        y = jnp.dot(xr[...], wr[...], preferred_element_type=jnp.float32)
        orf[...] = jnp.maximum(y + br[...], 0.0)

    return pl.pallas_call(
        body,
        grid=(n // tb,),
        in_specs=[
            pl.BlockSpec((tb, _D), lambda i: (i, 0)),
            pl.BlockSpec((_D, _D), lambda i: (0, 0)),
            pl.BlockSpec((1, _D), lambda i: (0, 0)),
        ],
        out_specs=pl.BlockSpec((tb, _D), lambda i: (i, 0)),
        out_shape=jax.ShapeDtypeStruct((n, _D), jnp.float32),
    )(x, w, b)


def _head_tc(psum, pmax, w0, b0, w1, b1, w2, b2, nsm):
    nt = w2.shape[1]

    def body(ps, pm, w0r, b0r, w1r, b1r, w2r, b2r, nr,
             o_out, o_lg, o_fp):
        sums = jnp.sum(ps[...], axis=0)
        mx = jnp.max(pm[...], axis=0)
        fp = jnp.tanh(jnp.concatenate([sums, mx], axis=1))
        h = jnp.dot(fp, w0r[...], preferred_element_type=jnp.float32)
        h = jnp.maximum(h + b0r[...], 0.0)
        h = jnp.dot(h, w1r[...], preferred_element_type=jnp.float32)
        h = jnp.maximum(h + b1r[...], 0.0)
        lg = jnp.dot(h, w2r[...], preferred_element_type=jnp.float32) + b2r[...]
        rmask = lax.broadcasted_iota(jnp.int32, (_NSEG, nt), 0) < nr[0, 0]
        lg = jnp.where(rmask, lg, 0.0)
        ii = lax.broadcasted_iota(jnp.int32, (nt, nt), 0)
        jj = lax.broadcasted_iota(jnp.int32, (nt, nt), 1)
        sw = (ii == (jj ^ 1)).astype(jnp.float32)
        lsw = jnp.dot(lg, sw, preferred_element_type=jnp.float32)
        m = jnp.maximum(lg, lsw)
        e = jnp.exp(lg - m)
        esw = jnp.dot(e, sw, preferred_element_type=jnp.float32)
        o_out[...] = e / (e + esw)
        o_lg[...] = lg
        o_fp[...] = fp

    return pl.pallas_call(
        body,
        out_shape=(
            jax.ShapeDtypeStruct((_NSEG, nt), jnp.float32),
            jax.ShapeDtypeStruct((_NSEG, nt), jnp.float32),
            jax.ShapeDtypeStruct((_NSEG, 2 * _D), jnp.float32),
        ),
    )(psum, pmax, w0, b0, w1, b1, w2, b2, nsm)


def kernel(atom_features, degree_slice, membership, n_samples,
           deg_adj_1, deg_adj_2, deg_adj_3, deg_adj_4, deg_adj_5,
           gc0_W, gc0_b, gc1_W, gc1_b, dense_W, dense_b,
           fd0_W, fd0_b, fd1_W, fd1_b, out_W, out_b):
    adjs = [deg_adj_1, deg_adj_2, deg_adj_3, deg_adj_4, deg_adj_5]
    sizes = [a.shape[0] for a in adjs]
    idx_ts = [a.T for a in adjs]

    def wcat(w):
        return jnp.stack([jnp.concatenate([w[2 * i], w[2 * i + 1]], axis=0)
                          for i in range(5)])

    w0, b0 = wcat(gc0_W), gc0_b[1:6]
    w1, b1 = wcat(gc1_W), gc1_b[1:6]

    rel0 = _nbr_reduce(atom_features, idx_ts, "sum")
    h1 = _conv_tc(atom_features, rel0, w0, b0, sizes)
    p1 = _nbr_reduce(h1, idx_ts, "max")
    rel1 = _nbr_reduce(p1, idx_ts, "sum")
    h2 = _conv_tc(p1, rel1, w1, b1, sizes)
    p2 = _nbr_reduce(h2, idx_ts, "max")
    dn = _dense_tc(p2, dense_W, dense_b.reshape(1, -1))
    ps, pm = _seg_partials(dn, membership)
    nsm = jnp.asarray(n_samples, jnp.int32).reshape(1, 1)
    out24, lg24, fp = _head_tc(
        ps, pm, fd0_W, fd0_b.reshape(1, -1), fd1_W, fd1_b.reshape(1, -1),
        out_W, out_b.reshape(1, -1), nsm)
    nt = out_W.shape[1]
    return (out24.reshape(_NSEG, nt // _NCLS, _NCLS),
            lg24.reshape(_NSEG, nt // _NCLS, _NCLS),
            fp)
